# baseline (speedup 1.0000x reference)
import jax
import jax.numpy as jnp
from jax.experimental import pallas as pl
from jax.experimental.pallas import tpu as pltpu

F32 = jnp.float32
BF16 = jnp.bfloat16

_TAP_D1 = tuple(32 * i + j for i in range(5) for j in range(5))
_TAP_D2 = tuple(16 * i + j for i in range(5) for j in range(5))


def _lenet_tile_kernel(x_ref, b1a_ref, w2b_ref, w1m_ref, w2m_ref, w3m_ref,
                       cb_ref, bf1_ref, bf2_ref, bf3_ref,
                       o_ref, p1_s, y2_s, z_s):
    for k in range(14):
        acc = jnp.dot(b1a_ref[...], x_ref[64 * k:64 * k + 256, :],
                      preferred_element_type=F32)
        for c in range(6):
            a = acc[64 * c:64 * c + 64, :].reshape(2, 32, 256)
            m = jnp.maximum(a[0], a[1]).reshape(16, 2, 256)
            m = jnp.maximum(m[:, 0, :], m[:, 1, :])
            m = jnp.maximum(m + cb_ref[0, c], 0.0)
            p1_s[240 * c + 16 * k:240 * c + 16 * k + 16, :] = m.astype(BF16)
    for c in range(6):
        p1_s[240 * c + 224:240 * c + 240, :] = jnp.zeros((16, 256), BF16)

    y2_s[...] = jnp.dot(w2b_ref[...], p1_s[...],
                        preferred_element_type=F32)

    for co in range(16):
        y = jnp.maximum(y2_s[160 * co:160 * co + 160, :] + cb_ref[1, co], 0.0)
        y = y.reshape(5, 2, 16, 256)
        y = jnp.maximum(y[:, 0], y[:, 1])
        y = y.reshape(5, 8, 2, 256)
        y = jnp.maximum(y[:, :, 0], y[:, :, 1])
        m = y[:, :5, :].reshape(25, 256)
        z_s[32 * co:32 * co + 32, :] = jnp.concatenate(
            [m, jnp.zeros((7, 256), F32)], axis=0).astype(BF16)

    h = jnp.dot(w1m_ref[...], z_s[...], preferred_element_type=F32)
    h = jnp.maximum(h + bf1_ref[...], 0.0).astype(BF16)
    h = jnp.dot(w2m_ref[...], h, preferred_element_type=F32)
    h = jnp.maximum(h + bf2_ref[...], 0.0).astype(BF16)
    o_ref[...] = (jnp.dot(w3m_ref[...], h, preferred_element_type=F32)
                  + bf3_ref[...])


def kernel(x, w1p, b1p, w2p, b2p, fc1p, bfc1, fc2p, bfc2, fc3p, bfc3):
    n = x.shape[0]
    nt = n // 256

    xt = jnp.pad(x.reshape(n, 1024).astype(BF16).T, ((0, 64), (0, 0)))

    lut1 = jnp.zeros((6, 134), F32).at[:, jnp.array(_TAP_D1)].set(w1p[:, :6].T)
    c_idx = jnp.arange(384) // 64
    p_idx = jnp.arange(384) % 64
    d1 = jnp.arange(256)[None, :] - p_idx[:, None]
    b1a = jnp.where((d1 >= 0) & (d1 <= 132),
                    lut1[c_idx[:, None], jnp.clip(d1, 0, 133)],
                    0.0).astype(BF16)

    w2lut = jnp.zeros((16, 6, 69), F32).at[:, :, jnp.array(_TAP_D2)].set(
        jnp.transpose(w2p[:, :6, :16], (2, 1, 0)))
    p2 = jnp.arange(2560) % 160
    co_idx = jnp.arange(2560) // 160
    qq = jnp.arange(1440) % 240
    ci_idx = jnp.arange(1440) // 240
    d2 = qq[None, :] - p2[:, None]
    w2b = jnp.where((d2 >= 0) & (d2 <= 68),
                    w2lut[co_idx[:, None], ci_idx[None, :], jnp.clip(d2, 0, 68)],
                    0.0).astype(BF16)

    w1m = jnp.zeros((128, 16, 32), F32).at[:120, :, :25].set(
        jnp.transpose(fc1p[:, :16, :120], (2, 1, 0))).reshape(128, 512)
    w1m = w1m.astype(BF16)
    w2m = fc2p.T.astype(BF16)
    w3m = fc3p.T[:16, :].astype(BF16)

    cb = jnp.zeros((2, 16), F32).at[0, :6].set(b1p[0, :6]).at[1, :].set(
        b2p[0, :16])
    bf1 = jnp.broadcast_to(bfc1.T, (128, 256))
    bf2 = jnp.broadcast_to(bfc2.T, (128, 256))
    bf3 = jnp.broadcast_to(bfc3.T[:16], (16, 256))

    out = pl.pallas_call(
        _lenet_tile_kernel,
        out_shape=jax.ShapeDtypeStruct((16, n), F32),
        grid=(nt,),
        in_specs=[
            pl.BlockSpec((1088, 256), lambda b: (0, b)),
            pl.BlockSpec((384, 256), lambda b: (0, 0)),
            pl.BlockSpec((2560, 1440), lambda b: (0, 0)),
            pl.BlockSpec((128, 512), lambda b: (0, 0)),
            pl.BlockSpec((128, 128), lambda b: (0, 0)),
            pl.BlockSpec((16, 128), lambda b: (0, 0)),
            pl.BlockSpec(memory_space=pltpu.SMEM),
            pl.BlockSpec((128, 256), lambda b: (0, 0)),
            pl.BlockSpec((128, 256), lambda b: (0, 0)),
            pl.BlockSpec((16, 256), lambda b: (0, 0)),
        ],
        out_specs=pl.BlockSpec((16, 256), lambda b: (0, b)),
        scratch_shapes=[
            pltpu.VMEM((1440, 256), BF16),
            pltpu.VMEM((2560, 256), F32),
            pltpu.VMEM((512, 256), BF16),
        ],
        compiler_params=pltpu.CompilerParams(
            dimension_semantics=("parallel",),
            vmem_limit_bytes=48 * 1024 * 1024,
        ),
    )(xt, b1a, w2b, w1m, w2m, w3m, cb, bf1, bf2, bf3)

    return out[:10, :].T


# speedup vs baseline: 149.1577x; 149.1577x over previous
import jax
import jax.numpy as jnp
from jax.experimental import pallas as pl
from jax.experimental.pallas import tpu as pltpu

F32 = jnp.float32
BF16 = jnp.bfloat16

_TAP_D1 = tuple(32 * i + j for i in range(5) for j in range(5))
_TAP_D2 = tuple(16 * i + j for i in range(5) for j in range(5))


def _lenet_tile_kernel(x_ref, b1a_ref, w2b_ref, w1m_ref, w2m_ref, w3m_ref,
                       cb_ref, bf1_ref, bf2_ref, bf3_ref,
                       o_ref, p1_s, y2_s, z_s):
    for k in range(14):
        acc = jnp.dot(b1a_ref[...], x_ref[64 * k:64 * k + 256, :],
                      preferred_element_type=F32)
        for c in range(6):
            a = acc[64 * c:64 * c + 64, :].reshape(2, 32, 256)
            m = jnp.maximum(a[0], a[1]).reshape(16, 2, 256)
            m = jnp.maximum(m[:, 0, :], m[:, 1, :])
            m = jnp.maximum(m + cb_ref[0, c], 0.0)
            p1_s[240 * c + 16 * k:240 * c + 16 * k + 16, :] = m.astype(BF16)
    for c in range(6):
        p1_s[240 * c + 224:240 * c + 240, :] = jnp.zeros((16, 256), BF16)

    y2_s[...] = jnp.dot(w2b_ref[...], p1_s[...],
                        preferred_element_type=F32)

    for co in range(16):
        y = jnp.maximum(y2_s[160 * co:160 * co + 160, :] + cb_ref[1, co], 0.0)
        y = y.reshape(5, 2, 16, 256)
        y = jnp.maximum(y[:, 0], y[:, 1])
        y = y.reshape(5, 8, 2, 256)
        y = jnp.maximum(y[:, :, 0], y[:, :, 1])
        m = y[:, :5, :].reshape(25, 256)
        z_s[32 * co:32 * co + 32, :] = jnp.concatenate(
            [m, jnp.zeros((7, 256), F32)], axis=0).astype(BF16)

    h = jnp.dot(w1m_ref[...], z_s[...], preferred_element_type=F32)
    h = jnp.maximum(h + bf1_ref[...], 0.0).astype(BF16)
    h = jnp.dot(w2m_ref[...], h, preferred_element_type=F32)
    h = jnp.maximum(h + bf2_ref[...], 0.0).astype(BF16)
    o_ref[...] = (jnp.dot(w3m_ref[...], h, preferred_element_type=F32)
                  + bf3_ref[...])


def kernel(x, w1p, b1p, w2p, b2p, fc1p, bfc1, fc2p, bfc2, fc3p, bfc3):
    n = x.shape[0]
    nt = n // 256

    xt = jnp.pad(x.reshape(n, 1024).astype(BF16).T, ((0, 64), (0, 0)))


    pat1 = jnp.zeros((6, 257), F32).at[:, jnp.array(_TAP_D1)].set(w1p[:, :6].T)
    b1a = jnp.broadcast_to(pat1[:, None, :], (6, 64, 257)).reshape(6, 64 * 257)
    b1a = b1a[:, :64 * 256].reshape(384, 256).astype(BF16)

    w2lut = jnp.zeros((16, 6, 240), F32).at[:, :, jnp.array(_TAP_D2)].set(
        jnp.transpose(w2p[:, :6, :16], (2, 1, 0)))
    pat2 = jnp.pad(w2lut.reshape(16, 1440), ((0, 0), (0, 1)))
    w2b = jnp.broadcast_to(pat2[:, None, :],
                           (16, 160, 1441)).reshape(16, 160 * 1441)
    w2b = w2b[:, :160 * 1440].reshape(2560, 1440).astype(BF16)

    w1m = jnp.zeros((128, 16, 32), F32).at[:120, :, :25].set(
        jnp.transpose(fc1p[:, :16, :120], (2, 1, 0))).reshape(128, 512)
    w1m = w1m.astype(BF16)
    w2m = fc2p.T.astype(BF16)
    w3m = fc3p.T[:16, :].astype(BF16)

    cb = jnp.zeros((2, 16), F32).at[0, :6].set(b1p[0, :6]).at[1, :].set(
        b2p[0, :16])
    bf1 = jnp.broadcast_to(bfc1.T, (128, 256))
    bf2 = jnp.broadcast_to(bfc2.T, (128, 256))
    bf3 = jnp.broadcast_to(bfc3.T[:16], (16, 256))

    out = pl.pallas_call(
        _lenet_tile_kernel,
        out_shape=jax.ShapeDtypeStruct((16, n), F32),
        grid=(nt,),
        in_specs=[
            pl.BlockSpec((1088, 256), lambda b: (0, b)),
            pl.BlockSpec((384, 256), lambda b: (0, 0)),
            pl.BlockSpec((2560, 1440), lambda b: (0, 0)),
            pl.BlockSpec((128, 512), lambda b: (0, 0)),
            pl.BlockSpec((128, 128), lambda b: (0, 0)),
            pl.BlockSpec((16, 128), lambda b: (0, 0)),
            pl.BlockSpec(memory_space=pltpu.SMEM),
            pl.BlockSpec((128, 256), lambda b: (0, 0)),
            pl.BlockSpec((128, 256), lambda b: (0, 0)),
            pl.BlockSpec((16, 256), lambda b: (0, 0)),
        ],
        out_specs=pl.BlockSpec((16, 256), lambda b: (0, b)),
        scratch_shapes=[
            pltpu.VMEM((1440, 256), BF16),
            pltpu.VMEM((2560, 256), F32),
            pltpu.VMEM((512, 256), BF16),
        ],
        compiler_params=pltpu.CompilerParams(
            dimension_semantics=("parallel",),
            vmem_limit_bytes=48 * 1024 * 1024,
        ),
    )(xt, b1a, w2b, w1m, w2m, w3m, cb, bf1, bf2, bf3)

    return out[:10, :].T


# speedup vs baseline: 175.3634x; 1.1757x over previous
import jax
import jax.numpy as jnp
from jax.experimental import pallas as pl
from jax.experimental.pallas import tpu as pltpu

F32 = jnp.float32
BF16 = jnp.bfloat16


def _lenet_tile_kernel(x_ref, b1a_ref, w2b_ref, w1m_ref, w2m_ref, w3m_ref,
                       bp1_ref, bp2_ref, bf1_ref, bf2_ref, bf3_ref,
                       o_ref, p1_s, y2_s):
    for u in range(14):
        out = jnp.dot(b1a_ref[...], x_ref[64 * u:64 * u + 256, :],
                      preferred_element_type=F32)
        o4 = out.reshape(4, 96, 256)
        m = jnp.maximum(jnp.maximum(o4[0], o4[1]),
                        jnp.maximum(o4[2], o4[3]))
        m = jnp.maximum(m + bp1_ref[...], 0.0).astype(BF16)
        for c in range(6):
            p1_s[240 * c + 16 * u:240 * c + 16 * u + 16, :] = \
                m[16 * c:16 * c + 16, :]
    for c in range(6):
        p1_s[240 * c + 224:240 * c + 240, :] = jnp.zeros((16, 256), BF16)

    y2_s[...] = jnp.dot(w2b_ref[...], p1_s[...],
                        preferred_element_type=F32)
    t01 = jnp.maximum(y2_s[0:640, :], y2_s[640:1280, :])
    t23 = jnp.maximum(y2_s[1280:1920, :], y2_s[1920:2560, :])
    z = jnp.maximum(jnp.maximum(t01, t23) + bp2_ref[...], 0.0)
    zb = z.astype(BF16)

    h = jnp.dot(w1m_ref[...], zb, preferred_element_type=F32)
    h = jnp.maximum(h + bf1_ref[...], 0.0).astype(BF16)
    h = jnp.dot(w2m_ref[...], h, preferred_element_type=F32)
    h = jnp.maximum(h + bf2_ref[...], 0.0).astype(BF16)
    o_ref[...] = (jnp.dot(w3m_ref[...], h, preferred_element_type=F32)
                  + bf3_ref[...])


def kernel(x, w1p, b1p, w2p, b2p, fc1p, bfc1, fc2p, bfc2, fc3p, bfc3):
    n = x.shape[0]
    nt = n // 256

    xt = jnp.pad(x.reshape(n, 1024).astype(BF16).T, ((0, 64), (0, 0)))

    w1v = w1p[:, :6]
    pat1 = jnp.zeros((2, 2, 6, 258), F32)
    for dy in range(2):
        for dx in range(2):
            d = tuple(32 * (dy + i) + (dx + j)
                      for i in range(5) for j in range(5))
            pat1 = pat1.at[dy, dx, :, jnp.array(d)].set(w1v)
    b1a = jnp.broadcast_to(pat1.reshape(24, 1, 258),
                           (24, 16, 258)).reshape(24, 16 * 258)
    b1a = b1a[:, :16 * 256].reshape(384, 256).astype(BF16)

    w2v = jnp.transpose(w2p[:, :6, :16], (0, 2, 1))
    pat2 = jnp.zeros((2, 2, 16, 6, 102), F32)
    for dy in range(2):
        for dx in range(2):
            d = tuple(16 * (dy + i) + (dx + j)
                      for i in range(5) for j in range(5))
            pat2 = pat2.at[dy, dx, :, :, jnp.array(d)].set(w2v)
    bt = jnp.broadcast_to(pat2.reshape(4, 16, 6, 1, 102),
                          (4, 16, 6, 8, 102)).reshape(4, 16, 6, 816)
    bt = bt[..., :800].reshape(4, 16, 6, 8, 100)
    sb = jnp.stack([jnp.pad(bt, ((0, 0),) * 3 + ((0, 0), (32 * s,
                                                          140 - 32 * s)))
                    for s in range(5)], axis=3)
    w2b = jnp.transpose(sb, (0, 1, 3, 4, 2, 5)).reshape(2560, 1440)
    w2b = w2b.astype(BF16)

    w1m = jnp.transpose(fc1p[:, :16, :120], (2, 1, 0)).reshape(120, 16, 5, 5)
    w1m = jnp.pad(w1m, ((0, 8), (0, 0), (0, 0), (0, 3))).reshape(128, 640)
    w1m = w1m.astype(BF16)
    w2m = fc2p.T.astype(BF16)
    w3m = fc3p.T[:16, :].astype(BF16)

    bp1 = jnp.broadcast_to(b1p[0, :6, None, None], (6, 16, 256)).reshape(
        96, 256)
    bp2 = jnp.broadcast_to(b2p[0, :16, None, None], (16, 40, 256)).reshape(
        640, 256)
    bf1 = jnp.broadcast_to(bfc1.T, (128, 256))
    bf2 = jnp.broadcast_to(bfc2.T, (128, 256))
    bf3 = jnp.broadcast_to(bfc3.T[:16], (16, 256))

    out = pl.pallas_call(
        _lenet_tile_kernel,
        out_shape=jax.ShapeDtypeStruct((16, n), F32),
        grid=(nt,),
        in_specs=[
            pl.BlockSpec((1088, 256), lambda b: (0, b)),
            pl.BlockSpec((384, 256), lambda b: (0, 0)),
            pl.BlockSpec((2560, 1440), lambda b: (0, 0)),
            pl.BlockSpec((128, 640), lambda b: (0, 0)),
            pl.BlockSpec((128, 128), lambda b: (0, 0)),
            pl.BlockSpec((16, 128), lambda b: (0, 0)),
            pl.BlockSpec((96, 256), lambda b: (0, 0)),
            pl.BlockSpec((640, 256), lambda b: (0, 0)),
            pl.BlockSpec((128, 256), lambda b: (0, 0)),
            pl.BlockSpec((128, 256), lambda b: (0, 0)),
            pl.BlockSpec((16, 256), lambda b: (0, 0)),
        ],
        out_specs=pl.BlockSpec((16, 256), lambda b: (0, b)),
        scratch_shapes=[
            pltpu.VMEM((1440, 256), BF16),
            pltpu.VMEM((2560, 256), F32),
        ],
        compiler_params=pltpu.CompilerParams(
            dimension_semantics=("parallel",),
            vmem_limit_bytes=48 * 1024 * 1024,
        ),
    )(xt, b1a, w2b, w1m, w2m, w3m, bp1, bp2, bf1, bf2, bf3)

    return out[:10, :].T


# speedup vs baseline: 175.8146x; 1.0026x over previous
import jax
import jax.numpy as jnp
from jax import lax
from jax.experimental import pallas as pl
from jax.experimental.pallas import tpu as pltpu

F32 = jnp.float32
BF16 = jnp.bfloat16


def _lenet_tile_kernel(x_ref, b1n_ref, w2b_ref, w1m_ref, w2m_ref, w3m_ref,
                       bp1_ref, bp2_ref, bf1_ref, bf2_ref, bf3_ref,
                       o_ref, p1_s, y2_s):
    xb = jnp.concatenate(
        [x_ref[...].astype(BF16), jnp.zeros((256, 128), BF16)], axis=1)

    for m in range(7):
        out = lax.dot_general(b1n_ref[...], xb[:, 128 * m:128 * m + 384],
                              (((1,), (1,)), ((), ())),
                              preferred_element_type=F32)
        o4 = out.reshape(4, 192, 256)
        mx = jnp.maximum(jnp.maximum(o4[0], o4[1]),
                         jnp.maximum(o4[2], o4[3]))
        mx = jnp.maximum(mx + bp1_ref[...], 0.0).astype(BF16)
        for c in range(6):
            p1_s[240 * c + 32 * m:240 * c + 32 * m + 32, :] = \
                mx[32 * c:32 * c + 32, :]
    for c in range(6):
        p1_s[240 * c + 224:240 * c + 240, :] = jnp.zeros((16, 256), BF16)

    y2_s[...] = jnp.dot(w2b_ref[...], p1_s[...],
                        preferred_element_type=F32)
    t01 = jnp.maximum(y2_s[0:400, :], y2_s[400:800, :])
    t23 = jnp.maximum(y2_s[800:1200, :], y2_s[1200:1600, :])
    z = jnp.maximum(jnp.maximum(t01, t23) + bp2_ref[...], 0.0)
    zb = z.astype(BF16)

    h = jnp.dot(w1m_ref[...], zb, preferred_element_type=F32)
    h = jnp.maximum(h + bf1_ref[...], 0.0).astype(BF16)
    h = jnp.dot(w2m_ref[...], h, preferred_element_type=F32)
    h = jnp.maximum(h + bf2_ref[...], 0.0).astype(BF16)
    o_ref[...] = (jnp.dot(w3m_ref[...], h, preferred_element_type=F32)
                  + bf3_ref[...])


def kernel(x, w1p, b1p, w2p, b2p, fc1p, bfc1, fc2p, bfc2, fc3p, bfc3):
    n = x.shape[0]
    nt = n // 256

    w1v = w1p[:, :6]
    pat1 = jnp.zeros((2, 2, 6, 198), F32)
    for dy in range(2):
        for dx in range(2):
            d = tuple(32 * (dy + i) + (dx + j)
                      for i in range(5) for j in range(5))
            pat1 = pat1.at[dy, dx, :, jnp.array(d)].set(w1v)
    bt1 = jnp.broadcast_to(pat1.reshape(4, 6, 1, 198),
                           (4, 6, 16, 198)).reshape(4, 6, 16 * 198)
    bt1 = bt1[:, :, :16 * 196].reshape(4, 6, 16, 196)
    b1n = jnp.stack([jnp.pad(bt1, ((0, 0), (0, 0), (0, 0), (0, 188))),
                     jnp.pad(bt1, ((0, 0), (0, 0), (0, 0), (64, 124)))],
                    axis=2).reshape(768, 384).astype(BF16)

    w2v = jnp.transpose(w2p[:, :6, :16], (0, 2, 1))
    pat2 = jnp.zeros((2, 2, 16, 6, 240), F32)
    for dy in range(2):
        for dx in range(2):
            d = tuple(16 * (dy + i) + (dx + j)
                      for i in range(5) for j in range(5))
            pat2 = pat2.at[dy, dx, :, :, jnp.array(d)].set(w2v)
    pp = jnp.pad(pat2.reshape(4, 16, 1440), ((0, 0), (0, 0), (0, 2)))
    bt2 = jnp.broadcast_to(pp.reshape(4, 16, 1, 1442),
                           (4, 16, 8, 1442)).reshape(4, 16, 8 * 1442)
    bt2 = bt2[:, :, :8 * 1440].reshape(4, 16, 8, 1440)
    w2b = jnp.stack(
        [jnp.pad(bt2, ((0, 0), (0, 0), (0, 0), (32 * s, 0)))[..., :1440]
         for s in range(5)], axis=2)
    w2b = w2b[:, :, :, :5, :].reshape(1600, 1440).astype(BF16)

    w1m = jnp.transpose(fc1p[:, :16, :120], (2, 1, 0)).reshape(120, 400)
    w1m = jnp.pad(w1m, ((0, 8), (0, 0))).astype(BF16)
    w2m = fc2p.T.astype(BF16)
    w3m = fc3p.T[:16, :].astype(BF16)

    bp1 = jnp.broadcast_to(b1p[0, :6, None, None], (6, 32, 256)).reshape(
        192, 256)
    bp2 = jnp.broadcast_to(b2p[0, :16, None, None], (16, 25, 256)).reshape(
        400, 256)
    bf1 = jnp.broadcast_to(bfc1.T, (128, 256))
    bf2 = jnp.broadcast_to(bfc2.T, (128, 256))
    bf3 = jnp.broadcast_to(bfc3.T[:16], (16, 256))

    out = pl.pallas_call(
        _lenet_tile_kernel,
        out_shape=jax.ShapeDtypeStruct((16, n), F32),
        grid=(nt,),
        in_specs=[
            pl.BlockSpec((256, 1024), lambda b: (b, 0)),
            pl.BlockSpec((768, 384), lambda b: (0, 0)),
            pl.BlockSpec((1600, 1440), lambda b: (0, 0)),
            pl.BlockSpec((128, 400), lambda b: (0, 0)),
            pl.BlockSpec((128, 128), lambda b: (0, 0)),
            pl.BlockSpec((16, 128), lambda b: (0, 0)),
            pl.BlockSpec((192, 256), lambda b: (0, 0)),
            pl.BlockSpec((400, 256), lambda b: (0, 0)),
            pl.BlockSpec((128, 256), lambda b: (0, 0)),
            pl.BlockSpec((128, 256), lambda b: (0, 0)),
            pl.BlockSpec((16, 256), lambda b: (0, 0)),
        ],
        out_specs=pl.BlockSpec((16, 256), lambda b: (0, b)),
        scratch_shapes=[
            pltpu.VMEM((1440, 256), BF16),
            pltpu.VMEM((1600, 256), F32),
        ],
        compiler_params=pltpu.CompilerParams(
            dimension_semantics=("parallel",),
            vmem_limit_bytes=48 * 1024 * 1024,
        ),
    )(x.reshape(n, 1024), b1n, w2b, w1m, w2m, w3m,
      bp1, bp2, bf1, bf2, bf3)

    return out[:10, :].T


# speedup vs baseline: 183.0839x; 1.0413x over previous
import jax
import jax.numpy as jnp
from jax import lax
from jax.experimental import pallas as pl
from jax.experimental.pallas import tpu as pltpu

F32 = jnp.float32
BF16 = jnp.bfloat16
NB = 512


def _lenet_tile_kernel(x_ref, b1n_ref, w2b_ref, w1m_ref, w2m_ref, w3m_ref,
                       bp1_ref, bp2_ref, bf1_ref, bf2_ref, bf3_ref,
                       o_ref, p1_s, y2_s):
    xb = jnp.concatenate(
        [x_ref[...].astype(BF16), jnp.zeros((NB, 128), BF16)], axis=1)

    for m in range(7):
        out = lax.dot_general(b1n_ref[...], xb[:, 128 * m:128 * m + 384],
                              (((1,), (1,)), ((), ())),
                              preferred_element_type=F32)
        o4 = out.reshape(4, 192, NB)
        mx = jnp.maximum(jnp.maximum(o4[0], o4[1]),
                         jnp.maximum(o4[2], o4[3]))
        mx = jnp.maximum(mx + bp1_ref[...], 0.0).astype(BF16)
        for c in range(6):
            p1_s[240 * c + 32 * m:240 * c + 32 * m + 32, :] = \
                mx[32 * c:32 * c + 32, :]
    for c in range(6):
        p1_s[240 * c + 224:240 * c + 240, :] = jnp.zeros((16, NB), BF16)

    y2_s[...] = jnp.dot(w2b_ref[...], p1_s[...],
                        preferred_element_type=F32)
    t01 = jnp.maximum(y2_s[0:400, :], y2_s[400:800, :])
    t23 = jnp.maximum(y2_s[800:1200, :], y2_s[1200:1600, :])
    z = jnp.maximum(jnp.maximum(t01, t23) + bp2_ref[...], 0.0)
    zb = z.astype(BF16)

    h = jnp.dot(w1m_ref[...], zb, preferred_element_type=F32)
    h = jnp.maximum(h + bf1_ref[...], 0.0).astype(BF16)
    h = jnp.dot(w2m_ref[...], h, preferred_element_type=F32)
    h = jnp.maximum(h + bf2_ref[...], 0.0).astype(BF16)
    o_ref[...] = (jnp.dot(w3m_ref[...], h, preferred_element_type=F32)
                  + bf3_ref[...])


def kernel(x, w1p, b1p, w2p, b2p, fc1p, bfc1, fc2p, bfc2, fc3p, bfc3):
    n = x.shape[0]
    nt = n // NB

    w1v = w1p[:, :6]
    pat1 = jnp.zeros((2, 2, 6, 198), F32)
    for dy in range(2):
        for dx in range(2):
            d = tuple(32 * (dy + i) + (dx + j)
                      for i in range(5) for j in range(5))
            pat1 = pat1.at[dy, dx, :, jnp.array(d)].set(w1v)
    bt1 = jnp.broadcast_to(pat1.reshape(4, 6, 1, 198),
                           (4, 6, 16, 198)).reshape(4, 6, 16 * 198)
    bt1 = bt1[:, :, :16 * 196].reshape(4, 6, 16, 196)
    b1n = jnp.stack([jnp.pad(bt1, ((0, 0), (0, 0), (0, 0), (0, 188))),
                     jnp.pad(bt1, ((0, 0), (0, 0), (0, 0), (64, 124)))],
                    axis=2).reshape(768, 384).astype(BF16)

    w2v = jnp.transpose(w2p[:, :6, :16], (0, 2, 1))
    pat2 = jnp.zeros((2, 2, 16, 6, 240), F32)
    for dy in range(2):
        for dx in range(2):
            d = tuple(16 * (dy + i) + (dx + j)
                      for i in range(5) for j in range(5))
            pat2 = pat2.at[dy, dx, :, :, jnp.array(d)].set(w2v)
    pp = jnp.pad(pat2.reshape(4, 16, 1440), ((0, 0), (0, 0), (0, 2)))
    bt2 = jnp.broadcast_to(pp.reshape(4, 16, 1, 1442),
                           (4, 16, 8, 1442)).reshape(4, 16, 8 * 1442)
    bt2 = bt2[:, :, :8 * 1440].reshape(4, 16, 8, 1440)
    w2b = jnp.stack(
        [jnp.pad(bt2, ((0, 0), (0, 0), (0, 0), (32 * s, 0)))[..., :1440]
         for s in range(5)], axis=2)
    w2b = w2b[:, :, :, :5, :].reshape(1600, 1440).astype(BF16)

    w1m = jnp.transpose(fc1p[:, :16, :120], (2, 1, 0)).reshape(120, 400)
    w1m = jnp.pad(w1m, ((0, 8), (0, 0))).astype(BF16)
    w2m = fc2p.T.astype(BF16)
    w3m = fc3p.T[:16, :].astype(BF16)

    bp1 = jnp.broadcast_to(b1p[0, :6, None, None], (6, 32, NB)).reshape(
        192, NB)
    bp2 = jnp.broadcast_to(b2p[0, :16, None, None], (16, 25, NB)).reshape(
        400, NB)
    bf1 = jnp.broadcast_to(bfc1.T, (128, NB))
    bf2 = jnp.broadcast_to(bfc2.T, (128, NB))
    bf3 = jnp.broadcast_to(bfc3.T[:16], (16, NB))

    out = pl.pallas_call(
        _lenet_tile_kernel,
        out_shape=jax.ShapeDtypeStruct((16, n), F32),
        grid=(nt,),
        in_specs=[
            pl.BlockSpec((NB, 1024), lambda b: (b, 0)),
            pl.BlockSpec((768, 384), lambda b: (0, 0)),
            pl.BlockSpec((1600, 1440), lambda b: (0, 0)),
            pl.BlockSpec((128, 400), lambda b: (0, 0)),
            pl.BlockSpec((128, 128), lambda b: (0, 0)),
            pl.BlockSpec((16, 128), lambda b: (0, 0)),
            pl.BlockSpec((192, NB), lambda b: (0, 0)),
            pl.BlockSpec((400, NB), lambda b: (0, 0)),
            pl.BlockSpec((128, NB), lambda b: (0, 0)),
            pl.BlockSpec((128, NB), lambda b: (0, 0)),
            pl.BlockSpec((16, NB), lambda b: (0, 0)),
        ],
        out_specs=pl.BlockSpec((16, NB), lambda b: (0, b)),
        scratch_shapes=[
            pltpu.VMEM((1440, NB), BF16),
            pltpu.VMEM((1600, NB), F32),
        ],
        compiler_params=pltpu.CompilerParams(
            dimension_semantics=("parallel",),
            vmem_limit_bytes=48 * 1024 * 1024,
        ),
    )(x.reshape(n, 1024), b1n, w2b, w1m, w2m, w3m,
      bp1, bp2, bf1, bf2, bf3)

    return out[:10, :].T


# speedup vs baseline: 292.2269x; 1.5961x over previous
import jax
import jax.numpy as jnp
from jax import lax
from jax.experimental import pallas as pl
from jax.experimental.pallas import tpu as pltpu

F32 = jnp.float32
BF16 = jnp.bfloat16
NB = 512


def _lenet_tile_kernel(x_ref, b1a_ref, b2c_ref, w1m_ref, w2p_ref, w3p_ref,
                       bp1_ref, bp2_ref, bf1_ref, bf2_ref, bf3_ref,
                       o_ref, p1_s, z_s):
    for u in range(14):
        out = jnp.dot(b1a_ref[...], x_ref[64 * u:64 * u + 256, :],
                      preferred_element_type=F32)
        o4 = out.reshape(4, 96, NB)
        mx = jnp.maximum(jnp.maximum(o4[0], o4[1]),
                         jnp.maximum(o4[2], o4[3]))
        mx = jnp.maximum(mx + bp1_ref[...], 0.0)
        p1_s[96 * u:96 * u + 96, :] = mx.astype(BF16)
    p1_s[1344:1536, :] = jnp.zeros((192, NB), BF16)

    for s in range(5):
        y = jnp.dot(b2c_ref[...], p1_s[192 * s:192 * s + 768, :],
                    preferred_element_type=F32)
        y4 = y.reshape(4, 80, NB)
        my = jnp.maximum(jnp.maximum(y4[0], y4[1]),
                         jnp.maximum(y4[2], y4[3]))
        my = jnp.maximum(my + bp2_ref[...], 0.0)
        z_s[80 * s:80 * s + 80, :] = my.astype(BF16)

    h = jnp.dot(w1m_ref[...], z_s[...], preferred_element_type=F32)
    h = jnp.maximum(h + bf1_ref[...], 0.0).astype(BF16)
    h = lax.dot_general(w2p_ref[...], h, (((0,), (0,)), ((), ())),
                        preferred_element_type=F32)
    h = jnp.maximum(h + bf2_ref[...], 0.0).astype(BF16)
    o = lax.dot_general(w3p_ref[...], h, (((0,), (0,)), ((), ())),
                        preferred_element_type=F32)
    o_ref[...] = o[:16, :] + bf3_ref[...]


def kernel(x, w1p, b1p, w2p, b2p, fc1p, bfc1, fc2p, bfc2, fc3p, bfc3):
    n = x.shape[0]
    nt = n // NB

    xt = jnp.pad(x.reshape(n, 1024).astype(BF16).T, ((0, 64), (0, 0)))

    w1v = w1p[:, :6]
    pat1 = jnp.zeros((2, 2, 6, 258), F32)
    for dy in range(2):
        for dx in range(2):
            d = tuple(32 * (dy + i) + (dx + j)
                      for i in range(5) for j in range(5))
            pat1 = pat1.at[dy, dx, :, jnp.array(d)].set(w1v)
    b1a = jnp.broadcast_to(pat1.reshape(4, 6, 1, 258),
                           (4, 6, 16, 258)).reshape(4, 6, 16 * 258)
    b1a = b1a[:, :, :16 * 256].reshape(4, 6, 16, 256)
    b1a = jnp.transpose(b1a, (0, 2, 1, 3)).reshape(384, 256).astype(BF16)

    w2f = w2p[:, :6, :16].reshape(150, 16)
    pat2 = jnp.zeros((2, 2, 16, 780), F32)
    for dy in range(2):
        for dx in range(2):
            d = tuple(6 * (16 * (dy + i) + (dx + j)) + ci
                      for i in range(5) for j in range(5) for ci in range(6))
            pat2 = pat2.at[dy, dx, :, jnp.array(d)].set(w2f)
    b2c = jnp.broadcast_to(pat2.reshape(4, 16, 1, 780),
                           (4, 16, 5, 780)).reshape(4, 16, 5 * 780)
    b2c = b2c[:, :, :5 * 768].reshape(320, 768).astype(BF16)

    w1m = jnp.transpose(fc1p[:, :16, :120], (2, 1, 0)).reshape(120, 16, 5, 5)
    w1m = jnp.transpose(w1m, (0, 2, 1, 3)).reshape(120, 400)
    w1m = jnp.pad(w1m, ((0, 8), (0, 0))).astype(BF16)

    bp1 = jnp.broadcast_to(b1p[0, :6][None, :, None], (16, 6, NB)).reshape(
        96, NB)
    bp2 = jnp.broadcast_to(b2p[0, :16][:, None, None], (16, 5, NB)).reshape(
        80, NB)
    bf1 = jnp.broadcast_to(bfc1.T, (128, NB))
    bf2 = jnp.broadcast_to(bfc2.T, (128, NB))
    bf3 = jnp.broadcast_to(bfc3.T[:16], (16, NB))

    out = pl.pallas_call(
        _lenet_tile_kernel,
        out_shape=jax.ShapeDtypeStruct((16, n), F32),
        grid=(nt,),
        in_specs=[
            pl.BlockSpec((1088, NB), lambda b: (0, b)),
            pl.BlockSpec((384, 256), lambda b: (0, 0)),
            pl.BlockSpec((320, 768), lambda b: (0, 0)),
            pl.BlockSpec((128, 400), lambda b: (0, 0)),
            pl.BlockSpec((128, 128), lambda b: (0, 0)),
            pl.BlockSpec((128, 128), lambda b: (0, 0)),
            pl.BlockSpec((96, NB), lambda b: (0, 0)),
            pl.BlockSpec((80, NB), lambda b: (0, 0)),
            pl.BlockSpec((128, NB), lambda b: (0, 0)),
            pl.BlockSpec((128, NB), lambda b: (0, 0)),
            pl.BlockSpec((16, NB), lambda b: (0, 0)),
        ],
        out_specs=pl.BlockSpec((16, NB), lambda b: (0, b)),
        scratch_shapes=[
            pltpu.VMEM((1536, NB), BF16),
            pltpu.VMEM((400, NB), BF16),
        ],
        compiler_params=pltpu.CompilerParams(
            dimension_semantics=("parallel",),
            vmem_limit_bytes=48 * 1024 * 1024,
        ),
    )(xt, b1a, b2c, w1m, fc2p.astype(BF16), fc3p.astype(BF16),
      bp1, bp2, bf1, bf2, bf3)

    return out[:10, :].T


# speedup vs baseline: 433.9335x; 1.4849x over previous
import numpy as np

import jax
import jax.numpy as jnp
from jax import lax
from jax.experimental import pallas as pl
from jax.experimental.pallas import tpu as pltpu

F32 = jnp.float32
BF16 = jnp.bfloat16
NB = 512

_E1 = np.zeros((25, 4 * 258), np.float32)
_E2 = np.zeros((150, 4 * 780), np.float32)
for _dy in range(2):
    for _dx in range(2):
        _off = 2 * _dy + _dx
        for _i in range(5):
            for _j in range(5):
                _tap = 5 * _i + _j
                _E1[_tap, 258 * _off + 32 * (_dy + _i) + (_dx + _j)] = 1.0
                for _ci in range(6):
                    _E2[6 * _tap + _ci,
                        780 * _off + 6 * (16 * (_dy + _i) + (_dx + _j))
                        + _ci] = 1.0


def _lenet_tile_kernel(x_ref, b1a_ref, b1z_ref, b2c_ref, w1m_ref, w2p_ref,
                       w3p_ref, bp1_ref, bp2_ref, bf1_ref, bf2_ref, bf3_ref,
                       o_ref, p1_s, z_s):
    for u in range(14):
        if u < 13:
            out = jnp.dot(b1a_ref[...], x_ref[64 * u:64 * u + 256, :],
                          preferred_element_type=F32)
        else:
            out = jnp.dot(b1z_ref[...], x_ref[832:1024, :],
                          preferred_element_type=F32)
        o4 = out.reshape(4, 96, NB)
        mx = jnp.maximum(jnp.maximum(o4[0], o4[1]),
                         jnp.maximum(o4[2], o4[3]))
        mx = jnp.maximum(mx + bp1_ref[...], 0.0)
        p1_s[96 * u:96 * u + 96, :] = mx.astype(BF16)
    p1_s[1344:1536, :] = jnp.zeros((192, NB), BF16)

    for s in range(5):
        y = jnp.dot(b2c_ref[...], p1_s[192 * s:192 * s + 768, :],
                    preferred_element_type=F32)
        y4 = y.reshape(4, 80, NB)
        my = jnp.maximum(jnp.maximum(y4[0], y4[1]),
                         jnp.maximum(y4[2], y4[3]))
        my = jnp.maximum(my + bp2_ref[...], 0.0)
        z_s[80 * s:80 * s + 80, :] = my.astype(BF16)

    h = jnp.dot(w1m_ref[...], z_s[...], preferred_element_type=F32)
    h = jnp.maximum(h + bf1_ref[...], 0.0).astype(BF16)
    h = lax.dot_general(w2p_ref[...], h, (((0,), (0,)), ((), ())),
                        preferred_element_type=F32)
    h = jnp.maximum(h + bf2_ref[...], 0.0).astype(BF16)
    o = lax.dot_general(w3p_ref[...], h, (((0,), (0,)), ((), ())),
                        preferred_element_type=F32)
    o_ref[...] = o[:16, :] + bf3_ref[...]


def kernel(x, w1p, b1p, w2p, b2p, fc1p, bfc1, fc2p, bfc2, fc3p, bfc3):
    n = x.shape[0]
    nt = n // NB

    xt = x.reshape(n, 1024).astype(BF16).T

    pat1 = lax.dot_general(w1p[:, :6], jnp.asarray(_E1),
                           (((0,), (0,)), ((), ())))
    pat1 = jnp.transpose(pat1.reshape(6, 4, 258), (1, 0, 2))
    b1a = jnp.broadcast_to(pat1.reshape(4, 6, 1, 258),
                           (4, 6, 16, 258)).reshape(4, 6, 16 * 258)
    b1a = b1a[:, :, :16 * 256].reshape(4, 6, 16, 256)
    b1a = jnp.transpose(b1a, (0, 2, 1, 3)).reshape(384, 256).astype(BF16)
    b1z = b1a[:, :192]

    pat2 = lax.dot_general(w2p[:, :6, :16].reshape(150, 16), jnp.asarray(_E2),
                           (((0,), (0,)), ((), ())))
    pat2 = jnp.transpose(pat2.reshape(16, 4, 780), (1, 0, 2))
    b2c = jnp.broadcast_to(pat2.reshape(4, 16, 1, 780),
                           (4, 16, 5, 780)).reshape(4, 16, 5 * 780)
    b2c = b2c[:, :, :5 * 768].reshape(320, 768).astype(BF16)

    w1m = jnp.transpose(fc1p[:, :16, :120], (2, 1, 0)).reshape(120, 16, 5, 5)
    w1m = jnp.transpose(w1m, (0, 2, 1, 3)).reshape(120, 400)
    w1m = jnp.pad(w1m, ((0, 8), (0, 0))).astype(BF16)

    bp1 = jnp.broadcast_to(b1p[0, :6][None, :, None], (16, 6, NB)).reshape(
        96, NB)
    bp2 = jnp.broadcast_to(b2p[0, :16][:, None, None], (16, 5, NB)).reshape(
        80, NB)
    bf1 = jnp.broadcast_to(bfc1.T, (128, NB))
    bf2 = jnp.broadcast_to(bfc2.T, (128, NB))
    bf3 = jnp.broadcast_to(bfc3.T[:16], (16, NB))

    out = pl.pallas_call(
        _lenet_tile_kernel,
        out_shape=jax.ShapeDtypeStruct((16, n), F32),
        grid=(nt,),
        in_specs=[
            pl.BlockSpec((1024, NB), lambda b: (0, b)),
            pl.BlockSpec((384, 256), lambda b: (0, 0)),
            pl.BlockSpec((384, 192), lambda b: (0, 0)),
            pl.BlockSpec((320, 768), lambda b: (0, 0)),
            pl.BlockSpec((128, 400), lambda b: (0, 0)),
            pl.BlockSpec((128, 128), lambda b: (0, 0)),
            pl.BlockSpec((128, 128), lambda b: (0, 0)),
            pl.BlockSpec((96, NB), lambda b: (0, 0)),
            pl.BlockSpec((80, NB), lambda b: (0, 0)),
            pl.BlockSpec((128, NB), lambda b: (0, 0)),
            pl.BlockSpec((128, NB), lambda b: (0, 0)),
            pl.BlockSpec((16, NB), lambda b: (0, 0)),
        ],
        out_specs=pl.BlockSpec((16, NB), lambda b: (0, b)),
        scratch_shapes=[
            pltpu.VMEM((1536, NB), BF16),
            pltpu.VMEM((400, NB), BF16),
        ],
        compiler_params=pltpu.CompilerParams(
            dimension_semantics=("parallel",),
            vmem_limit_bytes=48 * 1024 * 1024,
        ),
    )(xt, b1a, b1z, b2c, w1m, fc2p.astype(BF16), fc3p.astype(BF16),
      bp1, bp2, bf1, bf2, bf3)

    return out[:10, :].T


# speedup vs baseline: 437.8938x; 1.0091x over previous
import numpy as np

import jax
import jax.numpy as jnp
from jax import lax
from jax.experimental import pallas as pl
from jax.experimental.pallas import tpu as pltpu

F32 = jnp.float32
BF16 = jnp.bfloat16
NB = 1024

_E1 = np.zeros((25, 4 * 258), np.float32)
_E2 = np.zeros((150, 4 * 780), np.float32)
for _dy in range(2):
    for _dx in range(2):
        _off = 2 * _dy + _dx
        for _i in range(5):
            for _j in range(5):
                _tap = 5 * _i + _j
                _E1[_tap, 258 * _off + 32 * (_dy + _i) + (_dx + _j)] = 1.0
                for _ci in range(6):
                    _E2[6 * _tap + _ci,
                        780 * _off + 6 * (16 * (_dy + _i) + (_dx + _j))
                        + _ci] = 1.0


def _lenet_tile_kernel(x_ref, b1a_ref, b1z_ref, b2c_ref, w1m_ref, w2p_ref,
                       w3p_ref, bp1_ref, bp2_ref, bf1_ref, bf2_ref, bf3_ref,
                       o_ref, p1_s, z_s):
    for u in range(14):
        if u < 13:
            out = jnp.dot(b1a_ref[...], x_ref[64 * u:64 * u + 256, :],
                          preferred_element_type=F32)
        else:
            out = jnp.dot(b1z_ref[...], x_ref[832:1024, :],
                          preferred_element_type=F32)
        o4 = out.reshape(4, 96, NB)
        mx = jnp.maximum(jnp.maximum(o4[0], o4[1]),
                         jnp.maximum(o4[2], o4[3]))
        mx = jnp.maximum(mx + bp1_ref[...], 0.0)
        p1_s[96 * u:96 * u + 96, :] = mx.astype(BF16)
    p1_s[1344:1536, :] = jnp.zeros((192, NB), BF16)

    for s in range(5):
        y = jnp.dot(b2c_ref[...], p1_s[192 * s:192 * s + 768, :],
                    preferred_element_type=F32)
        y4 = y.reshape(4, 80, NB)
        my = jnp.maximum(jnp.maximum(y4[0], y4[1]),
                         jnp.maximum(y4[2], y4[3]))
        my = jnp.maximum(my + bp2_ref[...], 0.0)
        z_s[80 * s:80 * s + 80, :] = my.astype(BF16)

    h = jnp.dot(w1m_ref[...], z_s[...], preferred_element_type=F32)
    h = jnp.maximum(h + bf1_ref[...], 0.0).astype(BF16)
    h = lax.dot_general(w2p_ref[...], h, (((0,), (0,)), ((), ())),
                        preferred_element_type=F32)
    h = jnp.maximum(h + bf2_ref[...], 0.0).astype(BF16)
    o = lax.dot_general(w3p_ref[...], h, (((0,), (0,)), ((), ())),
                        preferred_element_type=F32)
    o_ref[...] = o[:16, :] + bf3_ref[...]


def kernel(x, w1p, b1p, w2p, b2p, fc1p, bfc1, fc2p, bfc2, fc3p, bfc3):
    n = x.shape[0]
    nt = n // NB

    xt = x.reshape(n, 1024).astype(BF16).T

    pat1 = lax.dot_general(w1p[:, :6], jnp.asarray(_E1),
                           (((0,), (0,)), ((), ())))
    pat1 = jnp.transpose(pat1.reshape(6, 4, 258), (1, 0, 2))
    b1a = jnp.broadcast_to(pat1.reshape(4, 6, 1, 258),
                           (4, 6, 16, 258)).reshape(4, 6, 16 * 258)
    b1a = b1a[:, :, :16 * 256].reshape(4, 6, 16, 256)
    b1a = jnp.transpose(b1a, (0, 2, 1, 3)).reshape(384, 256).astype(BF16)
    b1z = b1a[:, :192]

    pat2 = lax.dot_general(w2p[:, :6, :16].reshape(150, 16), jnp.asarray(_E2),
                           (((0,), (0,)), ((), ())))
    pat2 = jnp.transpose(pat2.reshape(16, 4, 780), (1, 0, 2))
    b2c = jnp.broadcast_to(pat2.reshape(4, 16, 1, 780),
                           (4, 16, 5, 780)).reshape(4, 16, 5 * 780)
    b2c = b2c[:, :, :5 * 768].reshape(320, 768).astype(BF16)

    w1m = jnp.transpose(fc1p[:, :16, :120], (2, 1, 0)).reshape(120, 16, 5, 5)
    w1m = jnp.transpose(w1m, (0, 2, 1, 3)).reshape(120, 400)
    w1m = jnp.pad(w1m, ((0, 8), (0, 0))).astype(BF16)

    bp1 = jnp.broadcast_to(b1p[0, :6][None, :, None], (16, 6, NB)).reshape(
        96, NB)
    bp2 = jnp.broadcast_to(b2p[0, :16][:, None, None], (16, 5, NB)).reshape(
        80, NB)
    bf1 = jnp.broadcast_to(bfc1.T, (128, NB))
    bf2 = jnp.broadcast_to(bfc2.T, (128, NB))
    bf3 = jnp.broadcast_to(bfc3.T[:16], (16, NB))

    out = pl.pallas_call(
        _lenet_tile_kernel,
        out_shape=jax.ShapeDtypeStruct((16, n), F32),
        grid=(nt,),
        in_specs=[
            pl.BlockSpec((1024, NB), lambda b: (0, b)),
            pl.BlockSpec((384, 256), lambda b: (0, 0)),
            pl.BlockSpec((384, 192), lambda b: (0, 0)),
            pl.BlockSpec((320, 768), lambda b: (0, 0)),
            pl.BlockSpec((128, 400), lambda b: (0, 0)),
            pl.BlockSpec((128, 128), lambda b: (0, 0)),
            pl.BlockSpec((128, 128), lambda b: (0, 0)),
            pl.BlockSpec((96, NB), lambda b: (0, 0)),
            pl.BlockSpec((80, NB), lambda b: (0, 0)),
            pl.BlockSpec((128, NB), lambda b: (0, 0)),
            pl.BlockSpec((128, NB), lambda b: (0, 0)),
            pl.BlockSpec((16, NB), lambda b: (0, 0)),
        ],
        out_specs=pl.BlockSpec((16, NB), lambda b: (0, b)),
        scratch_shapes=[
            pltpu.VMEM((1536, NB), BF16),
            pltpu.VMEM((400, NB), BF16),
        ],
        compiler_params=pltpu.CompilerParams(
            dimension_semantics=("parallel",),
            vmem_limit_bytes=48 * 1024 * 1024,
        ),
    )(xt, b1a, b1z, b2c, w1m, fc2p.astype(BF16), fc3p.astype(BF16),
      bp1, bp2, bf1, bf2, bf3)

    return out[:10, :].T
